# Initial kernel scaffold; baseline (speedup 1.0000x reference)
import jax
import jax.numpy as jnp
from jax.experimental import pallas as pl


def kernel(x, edge_index, W1, b1, gamma, beta, W2, b2):
    raise NotImplementedError("write your pallas kernel here")


# speedup vs baseline: 22.8036x; 22.8036x over previous
import functools

import jax
import jax.numpy as jnp
from jax import lax
from jax.experimental import pallas as pl
from jax.experimental.pallas import tpu as pltpu
from jax.experimental.pallas import tpu_sc as plsc

N = 10000
E = 320000
D = 128
EPS = 1e-5
NC = 2
NS = 16
NW = NC * NS
CHUNK = 128
WCH = 80
NCHT = NW * WCH
EPAD = NCHT * CHUNK
NDUMP = 16
N2 = N + NDUMP
FROWS = 640
LROWS = N - FROWS * (NS - 1)
FLUSH = 80
NPAD = 10240
DEGW = NPAD // NS

_f32 = jnp.float32
_i32 = jnp.int32


def _worker_id():
    return lax.axis_index("s") * NC + lax.axis_index("c")


def _load_my_chunks(hbm2d, buf, w):
    pltpu.sync_copy(hbm2d.at[pl.ds(WCH * w, WCH)], buf)


def _zero_vmem_2d(buf, nrows):
    zeros = jnp.zeros((16,), _f32)

    def body(r, _):
        for k in range(D // 16):
            buf[r, pl.ds(k * 16, 16)] = zeros
        return 0

    lax.fori_loop(0, nrows, body, 0)


def _sc_degree(dst2d):
    mesh = plsc.VectorSubcoreMesh(core_axis_name="c", subcore_axis_name="s")

    @functools.partial(
        pl.kernel,
        out_type=jax.ShapeDtypeStruct((NC * NPAD,), _f32),
        mesh=mesh,
        scratch_types=[
            pltpu.VMEM_SHARED((NPAD,), _f32),
            pltpu.VMEM((WCH, CHUNK), _i32),
            pltpu.VMEM((CHUNK,), _f32),
            pltpu.VMEM((DEGW,), _f32),
        ],
    )
    def deg_kernel(dst_hbm, out_hbm, deg_sh, didx, ones_v, stage):
        c = lax.axis_index("c")
        s = lax.axis_index("s")
        w = _worker_id()

        one16 = jnp.ones((16,), _f32)
        zero16 = jnp.zeros((16,), _f32)
        for k in range(CHUNK // 16):
            ones_v[pl.ds(k * 16, 16)] = one16

        def zbody(i, _):
            stage[pl.ds(i * 16, 16)] = zero16
            return 0
        lax.fori_loop(0, DEGW // 16, zbody, 0)

        pltpu.sync_copy(stage, deg_sh.at[pl.ds(s * DEGW, DEGW)])
        plsc.subcore_barrier()

        _load_my_chunks(dst_hbm, didx, w)

        def body(j, _):
            pltpu.sync_copy(ones_v, deg_sh.at[didx.at[j]], add=True)
            return 0
        lax.fori_loop(0, WCH, body, 0)
        plsc.subcore_barrier()

        pltpu.sync_copy(deg_sh.at[pl.ds(s * DEGW, DEGW)], stage)
        pltpu.sync_copy(stage, out_hbm.at[pl.ds(c * NPAD + s * DEGW, DEGW)])

    return deg_kernel(dst2d)


def _sc_aggregate(h, src2d, dst2d):
    mesh = plsc.VectorSubcoreMesh(core_axis_name="c", subcore_axis_name="s")

    @functools.partial(
        pl.kernel,
        out_type=jax.ShapeDtypeStruct((NC, N, D), _f32),
        mesh=mesh,
        scratch_types=[
            pltpu.VMEM_SHARED((N2, D), _f32),
            pltpu.VMEM((WCH, CHUNK), _i32),
            pltpu.VMEM((WCH, CHUNK), _i32),
            pltpu.VMEM((CHUNK, D), _f32),
            pltpu.VMEM((FLUSH, D), _f32),
            pltpu.SemaphoreType.DMA,
        ],
    )
    def agg_kernel(h_hbm, src_hbm, dst_hbm, out_hbm,
                   acc_sh, sidx, didx, rows, stage, sem):
        c = lax.axis_index("c")
        s = lax.axis_index("s")
        w = _worker_id()

        _zero_vmem_2d(stage, FLUSH)
        nfl = jnp.where(s < NS - 1, FROWS // FLUSH, LROWS // FLUSH)

        def zcopy(f, _):
            pltpu.sync_copy(stage, acc_sh.at[pl.ds(s * FROWS + f * FLUSH,
                                                   FLUSH)])
            return 0
        lax.fori_loop(0, nfl, zcopy, 0)
        plsc.subcore_barrier()

        _load_my_chunks(src_hbm, sidx, w)
        _load_my_chunks(dst_hbm, didx, w)

        def body(j, _):
            pltpu.async_copy(h_hbm.at[sidx.at[j]], rows, sem).wait()
            pltpu.sync_copy(rows, acc_sh.at[didx.at[j]], add=True)
            return 0
        lax.fori_loop(0, WCH, body, 0)
        plsc.subcore_barrier()

        def fcopy(f, _):
            r0 = s * FROWS + f * FLUSH
            pltpu.sync_copy(acc_sh.at[pl.ds(r0, FLUSH)], stage)
            pltpu.sync_copy(stage, out_hbm.at[c, pl.ds(r0, FLUSH)])
            return 0
        lax.fori_loop(0, nfl, fcopy, 0)

    return agg_kernel(h, src2d, dst2d)


BR = 2000
GRID = N // BR


def _dinv_block(degp_ref):
    v = degp_ref[...]
    dg = v[:, 0:1] + v[:, 1:2] + jnp.float32(1.0)
    return lax.rsqrt(dg)


def _tc1(x, W1, degp):
    def body(x_ref, w_ref, degp_ref, o_ref):
        i = pl.program_id(0)
        dinv = _dinv_block(degp_ref)
        h = lax.dot_general(x_ref[...], w_ref[...],
                            (((1,), (1,)), ((), ())),
                            preferred_element_type=_f32)
        o_ref[...] = h * dinv

    return pl.pallas_call(
        body,
        grid=(GRID,),
        in_specs=[
            pl.BlockSpec((BR, D), lambda i: (i, 0)),
            pl.BlockSpec((D, D), lambda i: (0, 0)),
            pl.BlockSpec((BR, NC), lambda i: (i, 0)),
        ],
        out_specs=pl.BlockSpec((BR, D), lambda i: (i, 0)),
        out_shape=jax.ShapeDtypeStruct((N, D), _f32),
    )(x, W1, degp)


def _tc2(acc1, h1p, degp, b1):
    def body(acc_ref, h_ref, degp_ref, b_ref, a_ref, s_ref):
        i = pl.program_id(0)
        dinv = _dinv_block(degp_ref)
        z = (acc_ref[0] + acc_ref[1] + h_ref[...]) * dinv + b_ref[...]
        a = jnp.maximum(z, jnp.float32(0.0))
        a_ref[...] = a
        part = jnp.concatenate(
            [jnp.sum(a, axis=0)[None, :], jnp.sum(a * a, axis=0)[None, :]], 0)

        @pl.when(i == 0)
        def _():
            s_ref[...] = part

        @pl.when(i > 0)
        def _():
            s_ref[...] += part

    return pl.pallas_call(
        body,
        grid=(GRID,),
        in_specs=[
            pl.BlockSpec((NC, BR, D), lambda i: (0, i, 0)),
            pl.BlockSpec((BR, D), lambda i: (i, 0)),
            pl.BlockSpec((BR, NC), lambda i: (i, 0)),
            pl.BlockSpec((1, D), lambda i: (0, 0)),
        ],
        out_specs=[
            pl.BlockSpec((BR, D), lambda i: (i, 0)),
            pl.BlockSpec((2, D), lambda i: (0, 0)),
        ],
        out_shape=[
            jax.ShapeDtypeStruct((N, D), _f32),
            jax.ShapeDtypeStruct((2, D), _f32),
        ],
    )(acc1, h1p, degp, b1)


def _tc3(a, sums, gamma, beta, degp, W2):
    def body(a_ref, s_ref, g_ref, be_ref, degp_ref, w_ref, o_ref):
        i = pl.program_id(0)
        dinv = _dinv_block(degp_ref)
        inv_n = jnp.float32(1.0 / N)
        mean = s_ref[0, :] * inv_n
        var = s_ref[1, :] * inv_n - mean * mean
        scale = lax.rsqrt(var + jnp.float32(EPS)) * g_ref[0, :]
        h2 = (a_ref[...] - mean[None, :]) * scale[None, :] + be_ref[...]
        h = lax.dot_general(h2, w_ref[...], (((1,), (1,)), ((), ())),
                            preferred_element_type=_f32)
        o_ref[...] = h * dinv

    return pl.pallas_call(
        body,
        grid=(GRID,),
        in_specs=[
            pl.BlockSpec((BR, D), lambda i: (i, 0)),
            pl.BlockSpec((2, D), lambda i: (0, 0)),
            pl.BlockSpec((1, D), lambda i: (0, 0)),
            pl.BlockSpec((1, D), lambda i: (0, 0)),
            pl.BlockSpec((BR, NC), lambda i: (i, 0)),
            pl.BlockSpec((D, D), lambda i: (0, 0)),
        ],
        out_specs=pl.BlockSpec((BR, D), lambda i: (i, 0)),
        out_shape=jax.ShapeDtypeStruct((N, D), _f32),
    )(a, sums, gamma, beta, degp, W2)


def _tc4(acc2, h2p, degp, b2):
    def body(acc_ref, h_ref, degp_ref, b_ref, o_ref):
        i = pl.program_id(0)
        dinv = _dinv_block(degp_ref)
        o_ref[...] = ((acc_ref[0] + acc_ref[1] + h_ref[...]) * dinv
                      + b_ref[...])

    return pl.pallas_call(
        body,
        grid=(GRID,),
        in_specs=[
            pl.BlockSpec((NC, BR, D), lambda i: (0, i, 0)),
            pl.BlockSpec((BR, D), lambda i: (i, 0)),
            pl.BlockSpec((BR, NC), lambda i: (i, 0)),
            pl.BlockSpec((1, D), lambda i: (0, 0)),
        ],
        out_specs=pl.BlockSpec((BR, D), lambda i: (i, 0)),
        out_shape=jax.ShapeDtypeStruct((N, D), _f32),
    )(acc2, h2p, degp, b2)


def kernel(x, edge_index, W1, b1, gamma, beta, W2, b2):
    ei = edge_index.astype(_i32)
    pad = jnp.arange(EPAD - E, dtype=_i32)
    src2d = jnp.concatenate([ei[0], pad % N]).reshape(NCHT, CHUNK)
    dst2d = jnp.concatenate([ei[1], N + pad % NDUMP]).reshape(NCHT, CHUNK)
    b1r = b1.reshape(1, D)
    b2r = b2.reshape(1, D)
    gr = gamma.reshape(1, D)
    br = beta.reshape(1, D)

    degp = _sc_degree(dst2d).reshape(NC, NPAD).T
    h1p = _tc1(x, W1, degp)
    acc1 = _sc_aggregate(h1p, src2d, dst2d)
    a, sums = _tc2(acc1, h1p, degp, b1r)
    h2p = _tc3(a, sums, gr, br, degp, W2)
    acc2 = _sc_aggregate(h2p, src2d, dst2d)
    return _tc4(acc2, h2p, degp, b2r)


# speedup vs baseline: 28.7266x; 1.2597x over previous
import functools

import jax
import jax.numpy as jnp
from jax import lax
from jax.experimental import pallas as pl
from jax.experimental.pallas import tpu as pltpu
from jax.experimental.pallas import tpu_sc as plsc

N = 10000
E = 320000
D = 128
EPS = 1e-5
NC = 2
NS = 16
NW = NC * NS
CHUNK = 128
WCH = 80
PCH = 40
NCHT = NW * WCH
EPAD = NCHT * CHUNK
NDUMP = 16
N2 = N + NDUMP
FROWS = 640
LROWS = N - FROWS * (NS - 1)
FLUSH = 80
NPAD = 10240
DEGW = NPAD // NS

_f32 = jnp.float32
_i32 = jnp.int32


def _worker_id():
    return lax.axis_index("s") * NC + lax.axis_index("c")


def _load_my_chunks(hbm2d, buf, w):
    pltpu.sync_copy(hbm2d.at[pl.ds(WCH * w, WCH)], buf)


def _zero_vmem_2d(buf, nrows):
    zeros = jnp.zeros((16,), _f32)

    def body(r, _):
        for k in range(D // 16):
            buf[r, pl.ds(k * 16, 16)] = zeros
        return 0

    lax.fori_loop(0, nrows, body, 0)


def _sc_degree(dst2d):
    mesh = plsc.VectorSubcoreMesh(core_axis_name="c", subcore_axis_name="s")

    @functools.partial(
        pl.kernel,
        out_type=jax.ShapeDtypeStruct((NC * NPAD,), _f32),
        mesh=mesh,
        scratch_types=[
            pltpu.VMEM_SHARED((NPAD,), _f32),
            pltpu.VMEM((WCH, CHUNK), _i32),
            pltpu.VMEM((CHUNK,), _f32),
            pltpu.VMEM((DEGW,), _f32),
        ],
    )
    def deg_kernel(dst_hbm, out_hbm, deg_sh, didx, ones_v, stage):
        c = lax.axis_index("c")
        s = lax.axis_index("s")
        w = _worker_id()

        one16 = jnp.ones((16,), _f32)
        zero16 = jnp.zeros((16,), _f32)
        for k in range(CHUNK // 16):
            ones_v[pl.ds(k * 16, 16)] = one16

        def zbody(i, _):
            stage[pl.ds(i * 16, 16)] = zero16
            return 0
        lax.fori_loop(0, DEGW // 16, zbody, 0)

        pltpu.sync_copy(stage, deg_sh.at[pl.ds(s * DEGW, DEGW)])
        plsc.subcore_barrier()

        _load_my_chunks(dst_hbm, didx, w)

        def body(j, _):
            pltpu.sync_copy(ones_v, deg_sh.at[didx.at[j]], add=True)
            return 0
        lax.fori_loop(0, WCH, body, 0)
        plsc.subcore_barrier()

        pltpu.sync_copy(deg_sh.at[pl.ds(s * DEGW, DEGW)], stage)
        pltpu.sync_copy(stage, out_hbm.at[pl.ds(c * NPAD + s * DEGW, DEGW)])

    return deg_kernel(dst2d)


def _sc_aggregate(h, src2d, dst2d):
    mesh = plsc.VectorSubcoreMesh(core_axis_name="c", subcore_axis_name="s")

    @functools.partial(
        pl.kernel,
        out_type=jax.ShapeDtypeStruct((NC, N, D), _f32),
        mesh=mesh,
        scratch_types=[
            pltpu.VMEM_SHARED((N2, D), _f32),
            pltpu.VMEM((PCH, CHUNK), _i32),
            pltpu.VMEM((PCH, CHUNK), _i32),
            pltpu.VMEM((CHUNK, D), _f32),
            pltpu.VMEM((CHUNK, D), _f32),
            pltpu.SemaphoreType.DMA,
            pltpu.SemaphoreType.DMA,
        ],
    )
    def agg_kernel(h_hbm, src_hbm, dst_hbm, out_hbm,
                   acc_sh, sidx, didx, rows0, rows1, sg0, sg1):
        c = lax.axis_index("c")
        s = lax.axis_index("s")
        w = _worker_id()

        stage = rows1.at[pl.ds(0, FLUSH)]
        _zero_vmem_2d(rows1, FLUSH)
        nfl = jnp.where(s < NS - 1, FROWS // FLUSH, LROWS // FLUSH)

        def zcopy(f, _):
            pltpu.sync_copy(stage, acc_sh.at[pl.ds(s * FROWS + f * FLUSH,
                                                   FLUSH)])
            return 0
        lax.fori_loop(0, nfl, zcopy, 0)
        plsc.subcore_barrier()

        rows = (rows0, rows1)
        sg = (sg0, sg1)

        def gather(j, b):
            pltpu.async_copy(h_hbm.at[sidx.at[j]], rows[b], sg[b])

        def gwait(b):
            pltpu.make_async_copy(h_hbm.at[pl.ds(0, CHUNK)], rows[b],
                                  sg[b]).wait()

        for ph in range(WCH // PCH):
            cb = WCH * w + ph * PCH
            pltpu.sync_copy(src_hbm.at[pl.ds(cb, PCH)], sidx)
            pltpu.sync_copy(dst_hbm.at[pl.ds(cb, PCH)], didx)

            gather(0, 0)

            def body(g, _):
                for b in (0, 1):
                    j = 2 * g + b
                    gwait(b)

                    @pl.when(j + 1 < PCH)
                    def _():
                        gather(j + 1, 1 - b)
                    pltpu.sync_copy(rows[b], acc_sh.at[didx.at[j]],
                                    add=True)
                return 0
            lax.fori_loop(0, PCH // 2, body, 0)
        plsc.subcore_barrier()

        def fcopy(f, _):
            r0 = s * FROWS + f * FLUSH
            pltpu.sync_copy(acc_sh.at[pl.ds(r0, FLUSH)], stage)
            pltpu.sync_copy(stage, out_hbm.at[c, pl.ds(r0, FLUSH)])
            return 0
        lax.fori_loop(0, nfl, fcopy, 0)

    return agg_kernel(h, src2d, dst2d)


BR = 2000
GRID = N // BR


def _dinv_block(degp_ref):
    v = degp_ref[...]
    dg = v[:, 0:1] + v[:, 1:2] + jnp.float32(1.0)
    return lax.rsqrt(dg)


def _tc1(x, W1, degp):
    def body(x_ref, w_ref, degp_ref, o_ref):
        i = pl.program_id(0)
        dinv = _dinv_block(degp_ref)
        h = lax.dot_general(x_ref[...], w_ref[...],
                            (((1,), (1,)), ((), ())),
                            preferred_element_type=_f32)
        o_ref[...] = h * dinv

    return pl.pallas_call(
        body,
        grid=(GRID,),
        in_specs=[
            pl.BlockSpec((BR, D), lambda i: (i, 0)),
            pl.BlockSpec((D, D), lambda i: (0, 0)),
            pl.BlockSpec((BR, NC), lambda i: (i, 0)),
        ],
        out_specs=pl.BlockSpec((BR, D), lambda i: (i, 0)),
        out_shape=jax.ShapeDtypeStruct((N, D), _f32),
    )(x, W1, degp)


def _tc2(acc1, h1p, degp, b1):
    def body(acc_ref, h_ref, degp_ref, b_ref, a_ref, s_ref):
        i = pl.program_id(0)
        dinv = _dinv_block(degp_ref)
        z = (acc_ref[0] + acc_ref[1] + h_ref[...]) * dinv + b_ref[...]
        a = jnp.maximum(z, jnp.float32(0.0))
        a_ref[...] = a
        part = jnp.concatenate(
            [jnp.sum(a, axis=0)[None, :], jnp.sum(a * a, axis=0)[None, :]], 0)

        @pl.when(i == 0)
        def _():
            s_ref[...] = part

        @pl.when(i > 0)
        def _():
            s_ref[...] += part

    return pl.pallas_call(
        body,
        grid=(GRID,),
        in_specs=[
            pl.BlockSpec((NC, BR, D), lambda i: (0, i, 0)),
            pl.BlockSpec((BR, D), lambda i: (i, 0)),
            pl.BlockSpec((BR, NC), lambda i: (i, 0)),
            pl.BlockSpec((1, D), lambda i: (0, 0)),
        ],
        out_specs=[
            pl.BlockSpec((BR, D), lambda i: (i, 0)),
            pl.BlockSpec((2, D), lambda i: (0, 0)),
        ],
        out_shape=[
            jax.ShapeDtypeStruct((N, D), _f32),
            jax.ShapeDtypeStruct((2, D), _f32),
        ],
    )(acc1, h1p, degp, b1)


def _tc3(a, sums, gamma, beta, degp, W2):
    def body(a_ref, s_ref, g_ref, be_ref, degp_ref, w_ref, o_ref):
        i = pl.program_id(0)
        dinv = _dinv_block(degp_ref)
        inv_n = jnp.float32(1.0 / N)
        mean = s_ref[0, :] * inv_n
        var = s_ref[1, :] * inv_n - mean * mean
        scale = lax.rsqrt(var + jnp.float32(EPS)) * g_ref[0, :]
        h2 = (a_ref[...] - mean[None, :]) * scale[None, :] + be_ref[...]
        h = lax.dot_general(h2, w_ref[...], (((1,), (1,)), ((), ())),
                            preferred_element_type=_f32)
        o_ref[...] = h * dinv

    return pl.pallas_call(
        body,
        grid=(GRID,),
        in_specs=[
            pl.BlockSpec((BR, D), lambda i: (i, 0)),
            pl.BlockSpec((2, D), lambda i: (0, 0)),
            pl.BlockSpec((1, D), lambda i: (0, 0)),
            pl.BlockSpec((1, D), lambda i: (0, 0)),
            pl.BlockSpec((BR, NC), lambda i: (i, 0)),
            pl.BlockSpec((D, D), lambda i: (0, 0)),
        ],
        out_specs=pl.BlockSpec((BR, D), lambda i: (i, 0)),
        out_shape=jax.ShapeDtypeStruct((N, D), _f32),
    )(a, sums, gamma, beta, degp, W2)


def _tc4(acc2, h2p, degp, b2):
    def body(acc_ref, h_ref, degp_ref, b_ref, o_ref):
        i = pl.program_id(0)
        dinv = _dinv_block(degp_ref)
        o_ref[...] = ((acc_ref[0] + acc_ref[1] + h_ref[...]) * dinv
                      + b_ref[...])

    return pl.pallas_call(
        body,
        grid=(GRID,),
        in_specs=[
            pl.BlockSpec((NC, BR, D), lambda i: (0, i, 0)),
            pl.BlockSpec((BR, D), lambda i: (i, 0)),
            pl.BlockSpec((BR, NC), lambda i: (i, 0)),
            pl.BlockSpec((1, D), lambda i: (0, 0)),
        ],
        out_specs=pl.BlockSpec((BR, D), lambda i: (i, 0)),
        out_shape=jax.ShapeDtypeStruct((N, D), _f32),
    )(acc2, h2p, degp, b2)


def kernel(x, edge_index, W1, b1, gamma, beta, W2, b2):
    ei = edge_index.astype(_i32)
    pad = jnp.arange(EPAD - E, dtype=_i32)
    src2d = jnp.concatenate([ei[0], pad % N]).reshape(NCHT, CHUNK)
    dst2d = jnp.concatenate([ei[1], N + pad % NDUMP]).reshape(NCHT, CHUNK)
    b1r = b1.reshape(1, D)
    b2r = b2.reshape(1, D)
    gr = gamma.reshape(1, D)
    br = beta.reshape(1, D)

    degp = _sc_degree(dst2d).reshape(NC, NPAD).T
    h1p = _tc1(x, W1, degp)
    acc1 = _sc_aggregate(h1p, src2d, dst2d)
    a, sums = _tc2(acc1, h1p, degp, b1r)
    h2p = _tc3(a, sums, gr, br, degp, W2)
    acc2 = _sc_aggregate(h2p, src2d, dst2d)
    return _tc4(acc2, h2p, degp, b2r)
